# baseline (speedup 1.0000x reference)
import functools
import numpy as np
import jax
import jax.numpy as jnp
from jax import lax
from jax.experimental import pallas as pl
from jax.experimental.pallas import tpu as pltpu
from jax.experimental.pallas import tpu_sc as plsc

N = 10000
E = 320000
NFEAT = 128
NHID = 8
NHEAD = 8
NCLASS = 40

N_PAD = 10240
E_TOT = E + N
E_PAD = 344064
EQ4 = E_PAD // 4
ET32 = E_PAD // 32
EQ6 = E_PAD // 6
C1 = 512
C3 = 512
NB = N_PAD // 1024

_f32 = jnp.float32
_i32 = jnp.int32


def _mesh():
    return plsc.VectorSubcoreMesh(core_axis_name="c", subcore_axis_name="s")


def _wid():
    return lax.axis_index("s") * 2 + lax.axis_index("c")


def _zero_ref(ref, n):
    z = jnp.zeros((16,), _f32)

    @plsc.parallel_loop(0, n // 16, unroll=8)
    def _(i):
        ref[pl.ds(i * 16, 16)] = z


def _tc1_body(x_ref, w_ref, ht_ref, hrow_ref):
    z = jnp.dot(x_ref[...], w_ref[...], preferred_element_type=_f32)
    hrow_ref[...] = z
    ht_ref[...] = z.T


def _tc1(x_pad, W1):
    return pl.pallas_call(
        _tc1_body,
        grid=(NB,),
        in_specs=[
            pl.BlockSpec((1024, NFEAT), lambda i: (i, 0)),
            pl.BlockSpec((NFEAT, 64), lambda i: (0, 0)),
        ],
        out_specs=[
            pl.BlockSpec((64, 1024), lambda i: (0, i)),
            pl.BlockSpec((1024, 64), lambda i: (i, 0)),
        ],
        out_shape=[
            jax.ShapeDtypeStruct((64, N_PAD), _f32),
            jax.ShapeDtypeStruct((N_PAD, 64), _f32),
        ],
    )(x_pad, W1)


def _tc_mid_body(part_ref, b_ref, w_ref, ht_ref, hrow_ref, *, fout, pad_to):
    g = jnp.sum(part_ref[...], axis=0) + b_ref[...]
    z = jnp.where(g > 0, g, jnp.exp(jnp.minimum(g, 0.0)) - 1.0)
    ht = lax.dot_general(w_ref[...], z, (((0,), (0,)), ((), ())),
                         preferred_element_type=_f32)
    ht_ref[...] = ht
    if pad_to > fout:
        ht = jnp.concatenate([ht, jnp.zeros((pad_to - fout, ht.shape[1]), _f32)],
                             axis=0)
    hrow_ref[...] = ht.T


def _tc_mid(part, b, W, fout, pad_to):
    ncopy = part.shape[0]
    body = functools.partial(_tc_mid_body, fout=fout, pad_to=pad_to)
    return pl.pallas_call(
        body,
        grid=(NB,),
        in_specs=[
            pl.BlockSpec((ncopy, 64, 1024), lambda i: (0, 0, i)),
            pl.BlockSpec((64, 1), lambda i: (0, 0)),
            pl.BlockSpec((64, fout), lambda i: (0, 0)),
        ],
        out_specs=[
            pl.BlockSpec((fout, 1024), lambda i: (0, i)),
            pl.BlockSpec((1024, pad_to), lambda i: (i, 0)),
        ],
        out_shape=[
            jax.ShapeDtypeStruct((fout, N_PAD), _f32),
            jax.ShapeDtypeStruct((N_PAD, pad_to), _f32),
        ],
    )(part, b, W)


def _tc_final_body(part_ref, b_ref, o_ref):
    g = jnp.sum(part_ref[...], axis=0) + b_ref[...]
    zt = g.T
    m = jnp.max(zt, axis=1, keepdims=True)
    ex = jnp.exp(zt - m)
    s = jnp.sum(ex, axis=1, keepdims=True)
    o_ref[...] = zt - m - jnp.log(s)


def _tc_final(part3, b3):
    return pl.pallas_call(
        _tc_final_body,
        grid=(NB,),
        in_specs=[
            pl.BlockSpec((6, NCLASS, 1024), lambda i: (0, 0, i)),
            pl.BlockSpec((NCLASS, 1), lambda i: (0, 0)),
        ],
        out_specs=pl.BlockSpec((1024, NCLASS), lambda i: (i, 0)),
        out_shape=jax.ShapeDtypeStruct((N_PAD, NCLASS), _f32),
    )(part3, b3)


def _sc_p1_l12(hT, srcg, dstg):
    nch = EQ4 // C1
    inv_sqrt_d = float(1.0 / np.sqrt(float(NHID)))

    @functools.partial(
        pl.kernel,
        out_type=(
            jax.ShapeDtypeStruct((32, N_PAD), _f32),
            jax.ShapeDtypeStruct((32, EQ4), _f32),
        ),
        mesh=_mesh(),
        compiler_params=pltpu.CompilerParams(needs_layout_passes=False, use_tc_tiling_on_sc=False),
        scratch_types=[
            pltpu.VMEM((8, N_PAD), _f32),
            pltpu.VMEM((C1,), _i32),
            pltpu.VMEM((C1,), _i32),
            pltpu.VMEM((N_PAD,), _f32),
            pltpu.VMEM((C1,), _f32),
        ],
    )
    def k(hT_h, src_h, dst_h, denp_h, exb_h, tab, svb, dvb, dacc, exs):
        wid = _wid()
        q = wid // 8
        h = wid % 8
        pltpu.sync_copy(hT_h.at[pl.ds(h * 8, 8), :], tab)
        _zero_ref(dacc, N_PAD)
        base0 = q * EQ4

        def chunk(t, _):
            gb = base0 + t * C1
            pltpu.sync_copy(src_h.at[pl.ds(gb, C1)], svb)
            pltpu.sync_copy(dst_h.at[pl.ds(gb, C1)], dvb)

            @plsc.parallel_loop(0, C1 // 16, unroll=2)
            def grp(j):
                sv = svb[pl.ds(j * 16, 16)]
                dv = dvb[pl.ds(j * 16, 16)]
                acc = jnp.zeros((16,), _f32)
                for r in range(8):
                    rs = jnp.full((16,), r, _i32)
                    kr = plsc.load_gather(tab, [rs, sv])
                    qr = plsc.load_gather(tab, [rs, dv])
                    acc = acc + kr * qr
                e = acc * inv_sqrt_d
                e = jnp.where(e >= 0.0, e, 0.2 * e)
                ex = jnp.exp(e)
                plsc.addupdate_scatter(dacc, [dv], ex)
                exs[pl.ds(j * 16, 16)] = ex
            pltpu.sync_copy(exs, exb_h.at[wid, pl.ds(t * C1, C1)])
            return 0

        lax.fori_loop(0, nch, chunk, 0)
        pltpu.sync_copy(dacc, denp_h.at[wid])

    return k(hT, srcg, dstg)


def _sc_p1_l3(hT3, srcg, dstg):
    nch = ET32 // C3
    inv_sqrt_d = float(1.0 / np.sqrt(float(NCLASS)))

    @functools.partial(
        pl.kernel,
        out_type=(
            jax.ShapeDtypeStruct((32, N_PAD), _f32),
            jax.ShapeDtypeStruct((32, ET32), _f32),
        ),
        mesh=_mesh(),
        compiler_params=pltpu.CompilerParams(needs_layout_passes=False, use_tc_tiling_on_sc=False),
        scratch_types=[
            pltpu.VMEM((8, N_PAD), _f32),
            pltpu.VMEM((C3,), _i32),
            pltpu.VMEM((C3,), _i32),
            pltpu.VMEM((ET32,), _f32),
            pltpu.VMEM((N_PAD,), _f32),
            pltpu.VMEM((C3,), _f32),
        ],
    )
    def k(hT_h, src_h, dst_h, denp_h, exb_h, tab, svb, dvb, eacc, dacc, exs):
        wid = _wid()
        base0 = wid * ET32
        _zero_ref(eacc, ET32)
        _zero_ref(dacc, N_PAD)

        for rg in range(5):
            pltpu.sync_copy(hT_h.at[pl.ds(rg * 8, 8), :], tab)
            last = rg == 4

            def chunk(t, _):
                gb = base0 + t * C3
                pltpu.sync_copy(src_h.at[pl.ds(gb, C3)], svb)
                pltpu.sync_copy(dst_h.at[pl.ds(gb, C3)], dvb)

                @plsc.parallel_loop(0, C3 // 16, unroll=2)
                def grp(j):
                    sv = svb[pl.ds(j * 16, 16)]
                    dv = dvb[pl.ds(j * 16, 16)]
                    acc = eacc[pl.ds(t * C3 + j * 16, 16)]
                    for r in range(8):
                        rs = jnp.full((16,), r, _i32)
                        kr = plsc.load_gather(tab, [rs, sv])
                        qr = plsc.load_gather(tab, [rs, dv])
                        acc = acc + kr * qr
                    if not last:
                        eacc[pl.ds(t * C3 + j * 16, 16)] = acc
                    else:
                        e = acc * inv_sqrt_d
                        e = jnp.where(e >= 0.0, e, 0.2 * e)
                        ex = jnp.exp(e)
                        plsc.addupdate_scatter(dacc, [dv], ex)
                        exs[pl.ds(j * 16, 16)] = ex
                if last:
                    pltpu.sync_copy(exs, exb_h.at[wid, pl.ds(t * C3, C3)])
                return 0

            lax.fori_loop(0, nch, chunk, 0)

        pltpu.sync_copy(dacc, denp_h.at[wid])

    return k(hT3, srcg, dstg)


def _make_sc_p2(nrows_out, nactive, base_len, mult, nden):
    nch = base_len // C1
    assert nch % 2 == 0
    idx_scr = [pltpu.VMEM((128,), _i32) for _ in range(8)]
    kbuf_scr = [pltpu.VMEM((128, 8), _f32) for _ in range(8)]

    @functools.partial(
        pl.kernel,
        out_type=jax.ShapeDtypeStruct((nrows_out, 8 * N_PAD), _f32),
        mesh=_mesh(),
        compiler_params=pltpu.CompilerParams(needs_layout_passes=False,
                                             use_tc_tiling_on_sc=False),
        scratch_types=[
            pltpu.VMEM((8 * N_PAD,), _f32),
            pltpu.VMEM((N_PAD,), _f32),
            pltpu.VMEM((N_PAD,), _f32),
            pltpu.VMEM((C1,), _i32),
            pltpu.VMEM((C1,), _i32),
            pltpu.VMEM((C1,), _f32),
            pltpu.VMEM((C1,), _i32),
            pltpu.VMEM((C1,), _i32),
            pltpu.VMEM((C1,), _f32),
        ] + idx_scr + kbuf_scr + [
            pltpu.SemaphoreType.DMA,
            pltpu.SemaphoreType.DMA,
        ],
    )
    def k(hrow_h, src_h, dst_h, denp_h, exb_h, outp_h,
          oacc, rden, stage, svA, dvA, exA, svB, dvB, exB,
          gA0, gA1, gA2, gA3, gB0, gB1, gB2, gB3,
          kA0, kA1, kA2, kA3, kB0, kB1, kB2, kB3, semA, semB):
        setA = (svA, dvA, exA, [gA0, gA1, gA2, gA3], [kA0, kA1, kA2, kA3],
                semA)
        setB = (svB, dvB, exB, [gB0, gB1, gB2, gB3], [kB0, kB1, kB2, kB3],
                semB)
        wid = _wid()

        @pl.when(wid < nactive)
        def _():
            if nden == 4:
                q = wid // 8
                h = wid % 8
            else:
                q = wid // 5
                h = wid % 5

            _zero_ref(rden, N_PAD)
            for w in range(nden):
                row = (w * 8 + h) if nden == 4 else w
                pltpu.sync_copy(denp_h.at[row], stage)

                @plsc.parallel_loop(0, N_PAD // 16, unroll=8)
                def radd(i):
                    rden[pl.ds(i * 16, 16)] = (rden[pl.ds(i * 16, 16)]
                                               + stage[pl.ds(i * 16, 16)])

            @plsc.parallel_loop(0, N_PAD // 16, unroll=8)
            def rinv(i):
                rden[pl.ds(i * 16, 16)] = 1.0 / (rden[pl.ds(i * 16, 16)]
                                                 + 1e-16)

            _zero_ref(oacc, 8 * N_PAD)

            base0 = q * base_len
            iota16 = lax.iota(_i32, 16)

            def fetch_idx(t, s):
                sv, dv, ex, _, _, _ = s
                gb = base0 + t * C1
                pltpu.sync_copy(src_h.at[pl.ds(gb, C1)], sv)
                pltpu.sync_copy(dst_h.at[pl.ds(gb, C1)], dv)
                if nden == 4:
                    pltpu.sync_copy(exb_h.at[wid, pl.ds(t * C1, C1)], ex)
                else:
                    pltpu.sync_copy(exb_h.at[pl.ds(gb, C1)], ex)

            def fire_gathers(s):
                sv, _, _, gbufs, kbufs, sem = s
                for u in range(4):
                    gu = gbufs[u]

                    @plsc.parallel_loop(0, 8, unroll=4)
                    def gcalc(j):
                        svv = sv[pl.ds(u * 128 + j * 16, 16)]
                        gu[pl.ds(j * 16, 16)] = svv * mult + h

                cps = [pltpu.make_async_copy(hrow_h.at[gbufs[u]], kbufs[u],
                                             sem) for u in range(4)]
                for cp in cps:
                    cp.start()

            def compute(s):
                _, dv_b, ex_b, gbufs, kbufs, sem = s
                for u in range(4):
                    pltpu.make_async_copy(hrow_h.at[gbufs[u]], kbufs[u],
                                          sem).wait()
                for u in range(4):
                    ku = kbufs[u]

                    @plsc.parallel_loop(0, 8, unroll=2)
                    def grp(j):
                        o16 = u * 128 + j * 16
                        dv = dv_b[pl.ds(o16, 16)]
                        exv = ex_b[pl.ds(o16, 16)]
                        rd = plsc.load_gather(rden, [dv])
                        alpha = exv * rd
                        rows = j * 16 + iota16
                        for r in range(8):
                            rs = jnp.full((16,), r, _i32)
                            kr = plsc.load_gather(ku, [rows, rs])
                            plsc.addupdate_scatter(oacc, [dv + r * N_PAD],
                                                   alpha * kr)

            def pair(p, _):
                t0 = 2 * p
                fetch_idx(t0, setA)
                fire_gathers(setA)

                @pl.when(p > 0)
                def _():
                    compute(setB)

                fetch_idx(t0 + 1, setB)
                fire_gathers(setB)
                compute(setA)
                return 0

            lax.fori_loop(0, nch // 2, pair, 0)
            compute(setB)
            pltpu.sync_copy(oacc, outp_h.at[wid])

    return k


def _sc_p2_l12(hrowf, srcg, dstg, denp, exb):
    k = _make_sc_p2(32, 32, EQ4, 8, 4)
    return k(hrowf, srcg, dstg, denp, exb)


def _sc_p2_l3(hrow3f, srcg, dstg, denp3, exb3f):
    k = _make_sc_p2(30, 30, EQ6, 6, 32)
    return k(hrow3f, srcg, dstg, denp3, exb3f)


def kernel(x, edge_index, W1, b1, W2, b2, W3, b3):
    loop = jnp.arange(N, dtype=edge_index.dtype)
    src = jnp.concatenate([edge_index[0], loop])
    dst = jnp.concatenate([edge_index[1], loop])
    pad_idx = jnp.full((E_PAD - E_TOT,), N_PAD - 1, dtype=src.dtype)
    srcg = jnp.concatenate([src, pad_idx])
    dstg = jnp.concatenate([dst, pad_idx])

    x_pad = jnp.pad(x, ((0, N_PAD - N), (0, 0)))

    hT1, hrow1 = _tc1(x_pad, W1)
    denp1, exb1 = _sc_p1_l12(hT1, srcg, dstg)
    outp1 = _sc_p2_l12(hrow1.reshape(N_PAD * 8, 8), srcg, dstg, denp1, exb1)
    part1 = outp1.reshape(4, 64, N_PAD)

    hT2, hrow2 = _tc_mid(part1, b1.reshape(64, 1), W2, 64, 64)
    denp2, exb2 = _sc_p1_l12(hT2, srcg, dstg)
    outp2 = _sc_p2_l12(hrow2.reshape(N_PAD * 8, 8), srcg, dstg, denp2, exb2)
    part2 = outp2.reshape(4, 64, N_PAD)

    hT3, hrow3 = _tc_mid(part2, b2.reshape(64, 1), W3, NCLASS, 48)
    denp3, exb3 = _sc_p1_l3(hT3, srcg, dstg)
    outp3 = _sc_p2_l3(hrow3.reshape(N_PAD * 6, 8), srcg, dstg, denp3,
                      exb3.reshape(E_PAD))
    part3 = outp3.reshape(6, NCLASS, N_PAD)

    out = _tc_final(part3, b3.reshape(NCLASS, 1))
    return out[:N]


# speedup vs baseline: 1.1570x; 1.1570x over previous
import functools
import numpy as np
import jax
import jax.numpy as jnp
from jax import lax
from jax.experimental import pallas as pl
from jax.experimental.pallas import tpu as pltpu
from jax.experimental.pallas import tpu_sc as plsc

N = 10000
E = 320000
NFEAT = 128
NHID = 8
NHEAD = 8
NCLASS = 40

N_PAD = 10240
E_TOT = E + N
E_PAD = 344064
EQ4 = E_PAD // 4
ET32 = E_PAD // 32
EQ6 = E_PAD // 6
C1 = 1024
C3 = 512
NB = N_PAD // 1024

_f32 = jnp.float32
_i32 = jnp.int32


def _mesh():
    return plsc.VectorSubcoreMesh(core_axis_name="c", subcore_axis_name="s")


def _wid():
    return lax.axis_index("s") * 2 + lax.axis_index("c")


def _zero_ref(ref, n):
    z = jnp.zeros((16,), _f32)

    @plsc.parallel_loop(0, n // 16, unroll=8)
    def _(i):
        ref[pl.ds(i * 16, 16)] = z


def _tc1_body(x_ref, w_ref, ht_ref, hrow_ref):
    z = jnp.dot(x_ref[...], w_ref[...], preferred_element_type=_f32)
    hrow_ref[...] = z
    ht_ref[...] = z.T


def _tc1(x_pad, W1):
    return pl.pallas_call(
        _tc1_body,
        grid=(NB,),
        in_specs=[
            pl.BlockSpec((1024, NFEAT), lambda i: (i, 0)),
            pl.BlockSpec((NFEAT, 64), lambda i: (0, 0)),
        ],
        out_specs=[
            pl.BlockSpec((64, 1024), lambda i: (0, i)),
            pl.BlockSpec((1024, 64), lambda i: (i, 0)),
        ],
        out_shape=[
            jax.ShapeDtypeStruct((64, N_PAD), _f32),
            jax.ShapeDtypeStruct((N_PAD, 64), _f32),
        ],
    )(x_pad, W1)


def _tc_mid_body(part_ref, b_ref, w_ref, ht_ref, hrow_ref, *, fout, pad_to):
    g = jnp.sum(part_ref[...], axis=0) + b_ref[...]
    z = jnp.where(g > 0, g, jnp.exp(jnp.minimum(g, 0.0)) - 1.0)
    ht = lax.dot_general(w_ref[...], z, (((0,), (0,)), ((), ())),
                         preferred_element_type=_f32)
    ht_ref[...] = ht
    if pad_to > fout:
        ht = jnp.concatenate([ht, jnp.zeros((pad_to - fout, ht.shape[1]), _f32)],
                             axis=0)
    hrow_ref[...] = ht.T


def _tc_mid(part, b, W, fout, pad_to):
    ncopy = part.shape[0]
    body = functools.partial(_tc_mid_body, fout=fout, pad_to=pad_to)
    return pl.pallas_call(
        body,
        grid=(NB,),
        in_specs=[
            pl.BlockSpec((ncopy, 64, 1024), lambda i: (0, 0, i)),
            pl.BlockSpec((64, 1), lambda i: (0, 0)),
            pl.BlockSpec((64, fout), lambda i: (0, 0)),
        ],
        out_specs=[
            pl.BlockSpec((fout, 1024), lambda i: (0, i)),
            pl.BlockSpec((1024, pad_to), lambda i: (i, 0)),
        ],
        out_shape=[
            jax.ShapeDtypeStruct((fout, N_PAD), _f32),
            jax.ShapeDtypeStruct((N_PAD, pad_to), _f32),
        ],
    )(part, b, W)


def _tc_final_body(part_ref, b_ref, o_ref):
    g = jnp.sum(part_ref[...], axis=0) + b_ref[...]
    zt = g.T
    m = jnp.max(zt, axis=1, keepdims=True)
    ex = jnp.exp(zt - m)
    s = jnp.sum(ex, axis=1, keepdims=True)
    o_ref[...] = zt - m - jnp.log(s)


def _tc_final(part3, b3):
    return pl.pallas_call(
        _tc_final_body,
        grid=(NB,),
        in_specs=[
            pl.BlockSpec((6, NCLASS, 1024), lambda i: (0, 0, i)),
            pl.BlockSpec((NCLASS, 1), lambda i: (0, 0)),
        ],
        out_specs=pl.BlockSpec((1024, NCLASS), lambda i: (i, 0)),
        out_shape=jax.ShapeDtypeStruct((N_PAD, NCLASS), _f32),
    )(part3, b3)


def _sc_p1_l12(hT, srcg, dstg):
    nch = EQ4 // C1
    inv_sqrt_d = float(1.0 / np.sqrt(float(NHID)))

    @functools.partial(
        pl.kernel,
        out_type=(
            jax.ShapeDtypeStruct((32, N_PAD), _f32),
            jax.ShapeDtypeStruct((32, EQ4), _f32),
        ),
        mesh=_mesh(),
        compiler_params=pltpu.CompilerParams(needs_layout_passes=False, use_tc_tiling_on_sc=False),
        scratch_types=[
            pltpu.VMEM((8, N_PAD), _f32),
            pltpu.VMEM((C1,), _i32),
            pltpu.VMEM((C1,), _i32),
            pltpu.VMEM((N_PAD,), _f32),
            pltpu.VMEM((C1,), _f32),
            pltpu.SemaphoreType.DMA,
        ],
    )
    def k(hT_h, src_h, dst_h, denp_h, exb_h, tab, svb, dvb, dacc, exs, semI):
        wid = _wid()
        q = wid // 8
        h = wid % 8
        pltpu.sync_copy(hT_h.at[pl.ds(h * 8, 8), :], tab)
        _zero_ref(dacc, N_PAD)
        base0 = q * EQ4

        def chunk(t, _):
            gb = base0 + t * C1
            icps = [pltpu.make_async_copy(src_h.at[pl.ds(gb, C1)], svb, semI),
                    pltpu.make_async_copy(dst_h.at[pl.ds(gb, C1)], dvb, semI)]
            for cp in icps:
                cp.start()
            for cp in icps:
                cp.wait()

            @plsc.parallel_loop(0, C1 // 16, unroll=2)
            def grp(j):
                sv = svb[pl.ds(j * 16, 16)]
                dv = dvb[pl.ds(j * 16, 16)]
                acc = jnp.zeros((16,), _f32)
                for r in range(8):
                    rs = jnp.full((16,), r, _i32)
                    kr = plsc.load_gather(tab, [rs, sv])
                    qr = plsc.load_gather(tab, [rs, dv])
                    acc = acc + kr * qr
                e = acc * inv_sqrt_d
                e = jnp.where(e >= 0.0, e, 0.2 * e)
                ex = jnp.exp(e)
                plsc.addupdate_scatter(dacc, [dv], ex)
                exs[pl.ds(j * 16, 16)] = ex
            pltpu.sync_copy(exs, exb_h.at[wid, pl.ds(t * C1, C1)])
            return 0

        lax.fori_loop(0, nch, chunk, 0)
        pltpu.sync_copy(dacc, denp_h.at[wid])

    return k(hT, srcg, dstg)


def _sc_p1_l3(hT3, srcg, dstg):
    nch = ET32 // C3
    inv_sqrt_d = float(1.0 / np.sqrt(float(NCLASS)))

    @functools.partial(
        pl.kernel,
        out_type=(
            jax.ShapeDtypeStruct((32, N_PAD), _f32),
            jax.ShapeDtypeStruct((32, ET32), _f32),
        ),
        mesh=_mesh(),
        compiler_params=pltpu.CompilerParams(needs_layout_passes=False, use_tc_tiling_on_sc=False),
        scratch_types=[
            pltpu.VMEM((8, N_PAD), _f32),
            pltpu.VMEM((C3,), _i32),
            pltpu.VMEM((C3,), _i32),
            pltpu.VMEM((ET32,), _f32),
            pltpu.VMEM((N_PAD,), _f32),
            pltpu.VMEM((C3,), _f32),
            pltpu.SemaphoreType.DMA,
        ],
    )
    def k(hT_h, src_h, dst_h, denp_h, exb_h, tab, svb, dvb, eacc, dacc, exs,
          semI):
        wid = _wid()
        base0 = wid * ET32
        _zero_ref(eacc, ET32)
        _zero_ref(dacc, N_PAD)

        for rg in range(5):
            pltpu.sync_copy(hT_h.at[pl.ds(rg * 8, 8), :], tab)
            last = rg == 4

            def chunk(t, _):
                gb = base0 + t * C3
                icps = [pltpu.make_async_copy(src_h.at[pl.ds(gb, C3)], svb,
                                              semI),
                        pltpu.make_async_copy(dst_h.at[pl.ds(gb, C3)], dvb,
                                              semI)]
                for cp in icps:
                    cp.start()
                for cp in icps:
                    cp.wait()

                @plsc.parallel_loop(0, C3 // 16, unroll=2)
                def grp(j):
                    sv = svb[pl.ds(j * 16, 16)]
                    dv = dvb[pl.ds(j * 16, 16)]
                    acc = eacc[pl.ds(t * C3 + j * 16, 16)]
                    for r in range(8):
                        rs = jnp.full((16,), r, _i32)
                        kr = plsc.load_gather(tab, [rs, sv])
                        qr = plsc.load_gather(tab, [rs, dv])
                        acc = acc + kr * qr
                    if not last:
                        eacc[pl.ds(t * C3 + j * 16, 16)] = acc
                    else:
                        e = acc * inv_sqrt_d
                        e = jnp.where(e >= 0.0, e, 0.2 * e)
                        ex = jnp.exp(e)
                        plsc.addupdate_scatter(dacc, [dv], ex)
                        exs[pl.ds(j * 16, 16)] = ex
                if last:
                    pltpu.sync_copy(exs, exb_h.at[wid, pl.ds(t * C3, C3)])
                return 0

            lax.fori_loop(0, nch, chunk, 0)

        pltpu.sync_copy(dacc, denp_h.at[wid])

    return k(hT3, srcg, dstg)


def _make_sc_p2(nrows_out, nactive, base_len, mult, nden):
    nch = base_len // C1
    nsub = C1 // 128
    idx_scr = [pltpu.VMEM((128,), _i32) for _ in range(nsub)]
    kbuf_scr = [pltpu.VMEM((128, 8), _f32) for _ in range(nsub)]

    @functools.partial(
        pl.kernel,
        out_type=jax.ShapeDtypeStruct((nrows_out, 8 * N_PAD), _f32),
        mesh=_mesh(),
        compiler_params=pltpu.CompilerParams(needs_layout_passes=False,
                                             use_tc_tiling_on_sc=False),
        scratch_types=[
            pltpu.VMEM((8 * N_PAD,), _f32),
            pltpu.VMEM((N_PAD,), _f32),
            pltpu.VMEM((N_PAD,), _f32),
            pltpu.VMEM((C1,), _i32),
            pltpu.VMEM((C1,), _i32),
            pltpu.VMEM((C1,), _f32),
        ] + idx_scr + kbuf_scr + [
            pltpu.SemaphoreType.DMA,
            pltpu.SemaphoreType.DMA,
        ],
    )
    def k(hrow_h, src_h, dst_h, denp_h, exb_h, outp_h,
          oacc, rden, stage, svb, dvb, exs, *scr):
        gbufs = list(scr[:nsub])
        kbufs = list(scr[nsub:2 * nsub])
        semI = scr[2 * nsub]
        semG = scr[2 * nsub + 1]
        wid = _wid()

        @pl.when(wid < nactive)
        def _():
            if nden == 4:
                q = wid // 8
                h = wid % 8
            else:
                q = wid // 5
                h = wid % 5

            _zero_ref(rden, N_PAD)
            for w in range(nden):
                row = (w * 8 + h) if nden == 4 else w
                pltpu.sync_copy(denp_h.at[row], stage)

                @plsc.parallel_loop(0, N_PAD // 16, unroll=8)
                def radd(i):
                    rden[pl.ds(i * 16, 16)] = (rden[pl.ds(i * 16, 16)]
                                               + stage[pl.ds(i * 16, 16)])

            @plsc.parallel_loop(0, N_PAD // 16, unroll=8)
            def rinv(i):
                rden[pl.ds(i * 16, 16)] = 1.0 / (rden[pl.ds(i * 16, 16)]
                                                 + 1e-16)

            _zero_ref(oacc, 8 * N_PAD)

            base0 = q * base_len
            iota16 = lax.iota(_i32, 16)

            def chunk(t, _):
                gb = base0 + t * C1
                icps = [pltpu.make_async_copy(src_h.at[pl.ds(gb, C1)], svb,
                                              semI),
                        pltpu.make_async_copy(dst_h.at[pl.ds(gb, C1)], dvb,
                                              semI)]
                if nden == 4:
                    icps.append(pltpu.make_async_copy(
                        exb_h.at[wid, pl.ds(t * C1, C1)], exs, semI))
                else:
                    icps.append(pltpu.make_async_copy(
                        exb_h.at[pl.ds(gb, C1)], exs, semI))
                for cp in icps:
                    cp.start()
                for cp in icps:
                    cp.wait()

                for u in range(nsub):
                    gu = gbufs[u]

                    @plsc.parallel_loop(0, 8, unroll=4)
                    def gcalc(j):
                        svv = svb[pl.ds(u * 128 + j * 16, 16)]
                        gu[pl.ds(j * 16, 16)] = svv * mult + h

                gcps = [pltpu.make_async_copy(hrow_h.at[gbufs[u]], kbufs[u],
                                              semG) for u in range(nsub)]
                for cp in gcps:
                    cp.start()
                for cp in gcps:
                    cp.wait()

                for u in range(nsub):
                    ku = kbufs[u]

                    @plsc.parallel_loop(0, 8, unroll=4)
                    def grp(j):
                        o16 = u * 128 + j * 16
                        dv = dvb[pl.ds(o16, 16)]
                        exv = exs[pl.ds(o16, 16)]
                        rd = plsc.load_gather(rden, [dv])
                        alpha = exv * rd
                        rows = j * 16 + iota16
                        for r in range(8):
                            rs = jnp.full((16,), r, _i32)
                            kr = plsc.load_gather(ku, [rows, rs])
                            plsc.addupdate_scatter(oacc, [dv + r * N_PAD],
                                                   alpha * kr)

                return 0

            lax.fori_loop(0, nch, chunk, 0)
            pltpu.sync_copy(oacc, outp_h.at[wid])

    return k


def _sc_p2_l12(hrowf, srcg, dstg, denp, exb):
    k = _make_sc_p2(32, 32, EQ4, 8, 4)
    return k(hrowf, srcg, dstg, denp, exb)


def _sc_p2_l3(hrow3f, srcg, dstg, denp3, exb3f):
    k = _make_sc_p2(30, 30, EQ6, 6, 32)
    return k(hrow3f, srcg, dstg, denp3, exb3f)


def kernel(x, edge_index, W1, b1, W2, b2, W3, b3):
    loop = jnp.arange(N, dtype=edge_index.dtype)
    src = jnp.concatenate([edge_index[0], loop])
    dst = jnp.concatenate([edge_index[1], loop])
    pad_idx = jnp.full((E_PAD - E_TOT,), N_PAD - 1, dtype=src.dtype)
    srcg = jnp.concatenate([src, pad_idx])
    dstg = jnp.concatenate([dst, pad_idx])

    x_pad = jnp.pad(x, ((0, N_PAD - N), (0, 0)))

    hT1, hrow1 = _tc1(x_pad, W1)
    denp1, exb1 = _sc_p1_l12(hT1, srcg, dstg)
    outp1 = _sc_p2_l12(hrow1.reshape(N_PAD * 8, 8), srcg, dstg, denp1, exb1)
    part1 = outp1.reshape(4, 64, N_PAD)

    hT2, hrow2 = _tc_mid(part1, b1.reshape(64, 1), W2, 64, 64)
    denp2, exb2 = _sc_p1_l12(hT2, srcg, dstg)
    outp2 = _sc_p2_l12(hrow2.reshape(N_PAD * 8, 8), srcg, dstg, denp2, exb2)
    part2 = outp2.reshape(4, 64, N_PAD)

    hT3, hrow3 = _tc_mid(part2, b2.reshape(64, 1), W3, NCLASS, 48)
    denp3, exb3 = _sc_p1_l3(hT3, srcg, dstg)
    outp3 = _sc_p2_l3(hrow3.reshape(N_PAD * 6, 8), srcg, dstg, denp3,
                      exb3.reshape(E_PAD))
    part3 = outp3.reshape(6, NCLASS, N_PAD)

    out = _tc_final(part3, b3.reshape(NCLASS, 1))
    return out[:N]
